# Initial kernel scaffold; baseline (speedup 1.0000x reference)
import jax
import jax.numpy as jnp
from jax.experimental import pallas as pl


def kernel(x, mask):
    raise NotImplementedError("write your pallas kernel here")


# speedup vs baseline: 2.0165x; 2.0165x over previous
import jax
import jax.numpy as jnp
from jax.experimental import pallas as pl
from jax.experimental.pallas import tpu as pltpu


def _block_cumsum(a):
    s = 1
    n = a.shape[0]
    while s < n:
        shifted = jnp.concatenate(
            [jnp.zeros((s, a.shape[1]), a.dtype), a[:-s]], axis=0)
        a = a + shifted
        s *= 2
    return a


def _body(x_ref, m_ref, o_ref, carry_ref):
    s = pl.program_id(2)

    @pl.when(s == 0)
    def _():
        carry_ref[...] = jnp.zeros_like(carry_ref)

    xm = jnp.where(m_ref[0], x_ref[0], jnp.zeros_like(x_ref[0]))
    c = _block_cumsum(xm)
    o_ref[0] = c + carry_ref[...]
    carry_ref[...] = carry_ref[...] + c[-1:, :]


def kernel(x, mask):
    B, S, D = x.shape
    S_BLK = 512
    D_BLK = 1024
    grid = (B, D // D_BLK, S // S_BLK)
    spec = pl.BlockSpec((1, S_BLK, D_BLK), lambda b, d, s: (b, s, d))
    return pl.pallas_call(
        _body,
        grid=grid,
        in_specs=[spec, spec],
        out_specs=spec,
        out_shape=jax.ShapeDtypeStruct((B, S, D), x.dtype),
        scratch_shapes=[pltpu.VMEM((1, D_BLK), jnp.float32)],
        compiler_params=pltpu.CompilerParams(
            dimension_semantics=("parallel", "parallel", "arbitrary")),
    )(x, mask)


# speedup vs baseline: 2.1599x; 1.0711x over previous
import jax
import jax.numpy as jnp
from jax import lax
from jax.experimental import pallas as pl
from jax.experimental.pallas import tpu as pltpu

S_BLK = 512
D_BLK = 1024
CHUNK = 256


def _body(x_ref, m_ref, o_ref, carry_ref):
    s = pl.program_id(2)

    @pl.when(s == 0)
    def _():
        carry_ref[...] = jnp.zeros_like(carry_ref)

    xm = jnp.where(m_ref[0], x_ref[0], jnp.zeros_like(x_ref[0]))
    hi = xm.astype(jnp.bfloat16)
    lo = (xm - hi.astype(jnp.float32)).astype(jnp.bfloat16)

    row = lax.broadcasted_iota(jnp.int32, (CHUNK, CHUNK), 0)
    col = lax.broadcasted_iota(jnp.int32, (CHUNK, CHUNK), 1)
    tri = (row >= col).astype(jnp.bfloat16)

    run = carry_ref[...]
    for c in range(S_BLK // CHUNK):
        h = hi[c * CHUNK:(c + 1) * CHUNK, :]
        l = lo[c * CHUNK:(c + 1) * CHUNK, :]
        cc = (jnp.dot(tri, h, preferred_element_type=jnp.float32)
              + jnp.dot(tri, l, preferred_element_type=jnp.float32)
              + run)
        o_ref[0, c * CHUNK:(c + 1) * CHUNK, :] = cc
        run = cc[-1:, :]
    carry_ref[...] = run


def kernel(x, mask):
    B, S, D = x.shape
    grid = (B, D // D_BLK, S // S_BLK)
    spec = pl.BlockSpec((1, S_BLK, D_BLK), lambda b, d, s: (b, s, d))
    return pl.pallas_call(
        _body,
        grid=grid,
        in_specs=[spec, spec],
        out_specs=spec,
        out_shape=jax.ShapeDtypeStruct((B, S, D), x.dtype),
        scratch_shapes=[pltpu.VMEM((1, D_BLK), jnp.float32)],
        compiler_params=pltpu.CompilerParams(
            dimension_semantics=("parallel", "parallel", "arbitrary")),
    )(x, mask)
